# baseline (speedup 1.0000x reference)
import functools

import jax
import jax.numpy as jnp
from jax import lax
from jax.experimental import pallas as pl
from jax.experimental.pallas import tpu as pltpu
from jax.experimental.pallas import tpu_sc as plsc

D = 128
B = 16384
NC = 2
NS = 16
NW = NC * NS
BPW = B // NW
CH = 64
NCH = BPW // CH


def _gather_body(mem_hbm, idx_hbm, out_hbm, idx_v, rows_v, gsem, wsem):
    wid = lax.axis_index("s") * NC + lax.axis_index("c")
    base = wid * BPW
    pltpu.sync_copy(idx_hbm.at[wid], idx_v)
    copies = [
        pltpu.async_copy(
            mem_hbm.at[idx_v.at[j]], rows_v.at[pl.ds(j * CH, CH)], gsem
        )
        for j in range(NCH)
    ]
    writes = []
    for j in range(NCH):
        copies[j].wait()
        writes.append(
            pltpu.async_copy(
                rows_v.at[pl.ds(j * CH, CH)],
                out_hbm.at[pl.ds(base + j * CH, CH)],
                wsem,
            )
        )
    for w in writes:
        w.wait()


@jax.jit
def kernel(memory, n_id):
    mesh = plsc.VectorSubcoreMesh(
        core_axis_name="c", subcore_axis_name="s", num_cores=NC, num_subcores=NS
    )
    gather = functools.partial(
        pl.kernel,
        out_type=jax.ShapeDtypeStruct((B, D), jnp.float32),
        mesh=mesh,
        scratch_types=[
            pltpu.VMEM((NCH, CH), jnp.int32),
            pltpu.VMEM((BPW, D), jnp.float32),
            pltpu.SemaphoreType.DMA,
            pltpu.SemaphoreType.DMA,
        ],
    )(_gather_body)
    idx = n_id.astype(jnp.int32).reshape(NW, NCH, CH)
    return gather(memory, idx)


# speedup vs baseline: 1.0183x; 1.0183x over previous
import functools

import jax
import jax.numpy as jnp
from jax import lax
from jax.experimental import pallas as pl
from jax.experimental.pallas import tpu as pltpu
from jax.experimental.pallas import tpu_sc as plsc

D = 128
B = 16384
NC = 2
NS = 16
NW = NC * NS
BPW = B // NW
CH = 128
NCH = BPW // CH


CHUNKS = ((0, 0, 64), (0, 64, 64), (1, 0, 128), (2, 0, 128),
          (3, 0, 64), (3, 64, 64))


def _gather_body(
    mem_hbm, idx_hbm, out_hbm, idx_v, rows_v, g0, g1, g2, g3, g4, g5, w0, w1
):
    wid = lax.axis_index("s") * NC + lax.axis_index("c")
    base = wid * BPW
    pltpu.sync_copy(idx_hbm.at[wid], idx_v)
    gsems = (g0, g1, g2, g3, g4, g5)
    wsems = (w0, w1)
    offs = []
    o = 0
    for (_, _, ln) in CHUNKS:
        offs.append(o)
        o += ln
    copies = [
        pltpu.async_copy(
            mem_hbm.at[idx_v.at[r].at[pl.ds(co, ln)]],
            rows_v.at[pl.ds(offs[j], ln)],
            gsems[j],
        )
        for j, (r, co, ln) in enumerate(CHUNKS)
    ]
    writes = []
    for j, (r, co, ln) in enumerate(CHUNKS):
        copies[j].wait()
        writes.append(
            pltpu.async_copy(
                rows_v.at[pl.ds(offs[j], ln)],
                out_hbm.at[pl.ds(base + offs[j], ln)],
                wsems[j % 2],
            )
        )
    for w in writes:
        w.wait()


@jax.jit
def kernel(memory, n_id):
    mesh = plsc.VectorSubcoreMesh(
        core_axis_name="c", subcore_axis_name="s", num_cores=NC, num_subcores=NS
    )
    gather = functools.partial(
        pl.kernel,
        out_type=jax.ShapeDtypeStruct((B, D), jnp.float32),
        mesh=mesh,
        scratch_types=[
            pltpu.VMEM((NCH, CH), jnp.int32),
            pltpu.VMEM((BPW, D), jnp.float32),
            pltpu.SemaphoreType.DMA,
            pltpu.SemaphoreType.DMA,
            pltpu.SemaphoreType.DMA,
            pltpu.SemaphoreType.DMA,
            pltpu.SemaphoreType.DMA,
            pltpu.SemaphoreType.DMA,
            pltpu.SemaphoreType.DMA,
            pltpu.SemaphoreType.DMA,
        ],
    )(_gather_body)
    idx = n_id.astype(jnp.int32).reshape(NW, NCH, CH)
    return gather(memory, idx)


# speedup vs baseline: 1.0220x; 1.0036x over previous
import functools

import jax
import jax.numpy as jnp
from jax import lax
from jax.experimental import pallas as pl
from jax.experimental.pallas import tpu as pltpu
from jax.experimental.pallas import tpu_sc as plsc

D = 128
B = 16384
NC = 2
NS = 16
NW = NC * NS
BPW = B // NW
CH = 128
NCH = BPW // CH


def _gather_body(
    mem_hbm, idx_hbm, out_hbm, idx_v, rows_v, g0, g1, g2, g3, w0, w1
):
    wid = lax.axis_index("s") * NC + lax.axis_index("c")
    base = wid * BPW
    pltpu.sync_copy(idx_hbm.at[wid], idx_v)
    gsems = (g0, g1, g2, g3)
    wsems = (w0, w1)
    copies = [
        pltpu.async_copy(
            mem_hbm.at[idx_v.at[j]], rows_v.at[pl.ds(j * CH, CH)], gsems[j]
        )
        for j in range(NCH)
    ]
    writes = []
    for j in range(NCH):
        copies[j].wait()
        writes.append(
            pltpu.async_copy(
                rows_v.at[pl.ds(j * CH, CH)],
                out_hbm.at[pl.ds(base + j * CH, CH)],
                wsems[j % 2],
            )
        )
    for w in writes:
        w.wait()


@jax.jit
def kernel(memory, n_id):
    mesh = plsc.VectorSubcoreMesh(
        core_axis_name="c", subcore_axis_name="s", num_cores=NC, num_subcores=NS
    )
    gather = functools.partial(
        pl.kernel,
        out_type=jax.ShapeDtypeStruct((B, D), jnp.float32),
        mesh=mesh,
        scratch_types=[
            pltpu.VMEM((NCH, CH), jnp.int32),
            pltpu.VMEM((BPW, D), jnp.float32),
            pltpu.SemaphoreType.DMA,
            pltpu.SemaphoreType.DMA,
            pltpu.SemaphoreType.DMA,
            pltpu.SemaphoreType.DMA,
            pltpu.SemaphoreType.DMA,
            pltpu.SemaphoreType.DMA,
        ],
    )(_gather_body)
    idx = n_id.astype(jnp.int32).reshape(NW, NCH, CH)
    return gather(memory, idx)
